# baseline (speedup 1.0000x reference)
import functools

import jax
import jax.numpy as jnp
from jax import lax
from jax.experimental import pallas as pl
from jax.experimental.pallas import tpu as pltpu
from jax.experimental.pallas import tpu_sc as plsc

_N = 12
_K = 4
_BETA = 0.8
_BIAS = 2.0
_L = 16


def _softplus16(a):
    t = jnp.exp(-jnp.abs(a))
    z = t / (t + 2.0)
    z2 = z * z
    p = 1.0 + z2 * (1.0 / 3.0 + z2 * (1.0 / 5.0 + z2 * (
        1.0 / 7.0 + z2 * (1.0 / 9.0 + z2 * (1.0 / 11.0 + z2 * (1.0 / 13.0))))))
    return jnp.maximum(a, 0.0) + 2.0 * z * p


def _sc_body(w_hbm, p_hbm, nz_hbm, out_hbm,
             w_v, p_v, nz_v, adj_v, rm_v, m_v, res_v,
             msh, sem):
    t = lax.axis_index("s")
    lanes = lax.iota(jnp.int32, _L)
    valid = lanes < _N
    lanes_c = jnp.where(valid, lanes, 0)

    @pl.when(t < _N)
    def _phase1():
        cp_w = pltpu.async_copy(w_hbm, w_v, sem)
        cp_p = pltpu.async_copy(p_hbm, p_v, sem)
        cp_n = pltpu.async_copy(nz_hbm, nz_v, sem)
        topm = lanes < _K
        row_idx = _N * t + lanes_c
        col_idx = t + _N * lanes_c
        cp_w.wait()
        cp_p.wait()
        cp_n.wait()
        w = plsc.load_gather(w_v, [row_idx])
        wt = plsc.load_gather(w_v, [col_idx])
        p = plsc.load_gather(p_v, [row_idx])
        nz = plsc.load_gather(nz_v, [row_idx])
        a = 0.5 * (w + wt) + _BETA * p - _BIAS
        sp = _softplus16(a)
        notdiag = lanes != t
        adjrow = jnp.where(jnp.logical_and(valid, notdiag), sp, 0.0)
        adj_v[...] = adjrow
        b = jnp.where(valid, adjrow + nz, -1.0)
        _, svals = plsc.sort_key_val(b, lanes, descending=True)
        rm_v[...] = jnp.zeros((_L,), jnp.float32)
        plsc.store_scatter(rm_v, [svals], jnp.ones((_L,), jnp.float32),
                           mask=topm)
        pltpu.sync_copy(rm_v, msh.at[pl.ds(_L * t, _L)])

    plsc.subcore_barrier()

    @pl.when(t < _N)
    def _phase2():
        pltpu.sync_copy(msh, m_v)
        colv = plsc.load_gather(m_v, [_L * lanes_c + t])
        sym = jnp.maximum(rm_v[...], colv)
        res = jnp.where(lanes == t, 0.0, adj_v[...] * sym)
        res_v[...] = res
        pltpu.sync_copy(res_v, out_hbm.at[t])


_sc_call = functools.partial(
    pl.kernel,
    mesh=plsc.VectorSubcoreMesh(core_axis_name="c", subcore_axis_name="s",
                                num_cores=1),
    out_type=jax.ShapeDtypeStruct((_N, _L), jnp.float32),
    scratch_types=[
        pltpu.VMEM((_N * _N,), jnp.float32),
        pltpu.VMEM((_N * _N,), jnp.float32),
        pltpu.VMEM((_N * _N,), jnp.float32),
        pltpu.VMEM((_L,), jnp.float32),
        pltpu.VMEM((_L,), jnp.float32),
        pltpu.VMEM((_N * _L,), jnp.float32),
        pltpu.VMEM((_L,), jnp.float32),
        pltpu.VMEM_SHARED((_N * _L,), jnp.float32),
        pltpu.SemaphoreType.DMA,
    ],
    compiler_params=pltpu.CompilerParams(needs_layout_passes=False, skip_device_barrier=True, disable_bounds_checks=True, disable_semaphore_checks=True),
)(_sc_body)


def kernel(W, P):
    W = W.astype(jnp.float32).reshape(_N * _N)
    P = P.astype(jnp.float32).reshape(_N * _N)
    noise = (jax.random.uniform(jax.random.key(1), (_N, _N),
                                dtype=jnp.float32) * 0.01).reshape(_N * _N)
    return _sc_call(W, P, noise)[:, :_N]


# speedup vs baseline: 1.0003x; 1.0003x over previous
import functools

import jax
import jax.numpy as jnp
from jax import lax
from jax.experimental import pallas as pl
from jax.experimental.pallas import tpu as pltpu
from jax.experimental.pallas import tpu_sc as plsc

_N = 12
_K = 4
_BETA = 0.8
_BIAS = 2.0
_L = 16


def _softplus16(a):
    t = jnp.exp(-jnp.abs(a))
    z = t / (t + 2.0)
    z2 = z * z
    p = 1.0 + z2 * (1.0 / 3.0 + z2 * (1.0 / 5.0 + z2 * (
        1.0 / 7.0 + z2 * (1.0 / 9.0 + z2 * (1.0 / 11.0 + z2 * (1.0 / 13.0))))))
    return jnp.maximum(a, 0.0) + 2.0 * z * p


def _sc_body(w_hbm, p_hbm, nz_hbm, out_hbm,
             w_v, p_v, nz_v, adj_v, rm_v, m_v, res_v,
             msh, sem):
    t = lax.axis_index("s")
    lanes = lax.iota(jnp.int32, _L)
    valid = lanes < _N
    lanes_c = jnp.where(valid, lanes, 0)

    @pl.when(t < _N)
    def _phase1():
        cp_w = pltpu.async_copy(w_hbm, w_v, sem)
        cp_p = pltpu.async_copy(p_hbm, p_v, sem)
        cp_n = pltpu.async_copy(nz_hbm, nz_v, sem)
        topm = lanes < _K
        row_idx = _N * t + lanes_c
        col_idx = t + _N * lanes_c
        cp_w.wait()
        cp_p.wait()
        cp_n.wait()
        w = plsc.load_gather(w_v, [row_idx])
        wt = plsc.load_gather(w_v, [col_idx])
        p = plsc.load_gather(p_v, [row_idx])
        nz = plsc.load_gather(nz_v, [row_idx])
        a = 0.5 * (w + wt) + _BETA * p - _BIAS
        sp = _softplus16(a)
        notdiag = lanes != t
        adjrow = jnp.where(jnp.logical_and(valid, notdiag), sp, 0.0)
        adj_v[...] = adjrow
        b = jnp.where(valid, adjrow + nz, -1.0)
        _, svals = plsc.sort_key_val(b, lanes, descending=True)
        rm_v[...] = jnp.zeros((_L,), jnp.float32)
        plsc.store_scatter(rm_v, [svals], jnp.ones((_L,), jnp.float32),
                           mask=topm)
        pltpu.sync_copy(rm_v, msh.at[pl.ds(_L * t, _L)])

    plsc.subcore_barrier()

    @pl.when(t < _N)
    def _phase2():
        pltpu.sync_copy(msh, m_v)
        colv = plsc.load_gather(m_v, [_L * lanes_c + t])
        sym = jnp.maximum(rm_v[...], colv)
        res = jnp.where(lanes == t, 0.0, adj_v[...] * sym)
        res_v[...] = res
        pltpu.sync_copy(res_v, out_hbm.at[t])


_sc_call = functools.partial(
    pl.kernel,
    mesh=plsc.VectorSubcoreMesh(core_axis_name="c", subcore_axis_name="s",
                                num_cores=1),
    out_type=jax.ShapeDtypeStruct((_N, _L), jnp.float32),
    scratch_types=[
        pltpu.VMEM((_N * _N,), jnp.float32),
        pltpu.VMEM((_N * _N,), jnp.float32),
        pltpu.VMEM((_N * _N,), jnp.float32),
        pltpu.VMEM((_L,), jnp.float32),
        pltpu.VMEM((_L,), jnp.float32),
        pltpu.VMEM((_N * _L,), jnp.float32),
        pltpu.VMEM((_L,), jnp.float32),
        pltpu.VMEM_SHARED((_N * _L,), jnp.float32),
        pltpu.SemaphoreType.DMA,
    ],
    compiler_params=pltpu.CompilerParams(needs_layout_passes=False),
)(_sc_body)


def kernel(W, P):
    W = W.astype(jnp.float32).reshape(_N * _N)
    P = P.astype(jnp.float32).reshape(_N * _N)
    noise = (jax.random.uniform(jax.random.key(1), (_N, _N),
                                dtype=jnp.float32) * 0.01).reshape(_N * _N)
    return _sc_call(W, P, noise)[:, :_N]
